# Initial kernel scaffold; baseline (speedup 1.0000x reference)
import jax
import jax.numpy as jnp
from jax.experimental import pallas as pl


def kernel(hidden_states, Wq, Wk, Wv, Wo, bank_weights, memories, memory_norms):
    raise NotImplementedError("write your pallas kernel here")


# speedup vs baseline: 2.6052x; 2.6052x over previous
import functools

import jax
import jax.numpy as jnp
from jax.experimental import pallas as pl
from jax.experimental.pallas import tpu as pltpu

EPS = 1e-6


def _phase_kernel(x_ref, w_ref, memsc_ref, mem0_ref, nrmq_ref, knrm_ref,
                  out_ref, mem_part_ref, norm_part_ref,
                  comb_ref, sigk_ref, rk_ref, *, n_heads, d_head, n_banks):
    i = pl.program_id(1)
    j = pl.program_id(2)
    H, D, NB = n_heads, d_head, n_banks

    @pl.when(j == 0)
    def _q_phase():
        q = jnp.dot(x_ref[...], w_ref[0], preferred_element_type=jnp.float32)
        sq = jnp.where(q > 0, q + 1.0, jnp.exp(q))
        sqb = sq.astype(jnp.bfloat16)
        nrm = jnp.dot(sqb, nrmq_ref[...], preferred_element_type=jnp.float32)
        inv = 1.0 / jnp.maximum(nrm, EPS)
        for h in range(H):
            sq_h = sq[:, h * D:(h + 1) * D]
            parts = [
                (sq_h * inv[:, h * NB + n:h * NB + n + 1]).astype(jnp.bfloat16)
                for n in range(NB)
            ]
            a = jnp.concatenate(parts, axis=1)
            comb = jnp.dot(a, memsc_ref[h], preferred_element_type=jnp.float32)
            comb_ref[:, h * D:(h + 1) * D] = comb.astype(jnp.bfloat16)

    @pl.when(j == 1)
    def _k_phase():
        k = jnp.dot(x_ref[...], w_ref[0], preferred_element_type=jnp.float32)
        sk = jnp.where(k > 0, k + 1.0, jnp.exp(k))
        skb = sk.astype(jnp.bfloat16)
        sigk_ref[...] = skb
        kn = jnp.dot(skb, knrm_ref[...], preferred_element_type=jnp.float32)
        ikn = 1.0 / jnp.maximum(kn, EPS)
        for h in range(H):
            retr = jnp.dot(skb[:, h * D:(h + 1) * D], mem0_ref[h],
                           preferred_element_type=jnp.float32)
            rk_ref[:, h * D:(h + 1) * D] = retr * ikn[:, h:h + 1]
        part = jnp.sum(sk, axis=0)[None, None, :]

        @pl.when(i == 0)
        def _():
            norm_part_ref[...] = part

        @pl.when(i != 0)
        def _():
            norm_part_ref[...] += part

    @pl.when(j == 2)
    def _v_phase():
        v = jnp.dot(x_ref[...], w_ref[0], preferred_element_type=jnp.float32)
        dv = (v - rk_ref[...]).astype(jnp.bfloat16)
        sk = sigk_ref[...]
        pms = [
            jax.lax.dot_general(
                sk[:, h * D:(h + 1) * D], dv[:, h * D:(h + 1) * D],
                dimension_numbers=(((0,), (0,)), ((), ())),
                preferred_element_type=jnp.float32)
            for h in range(H)
        ]

        @pl.when(i == 0)
        def _():
            for h in range(H):
                mem_part_ref[0, h] = pms[h]

        @pl.when(i != 0)
        def _():
            for h in range(H):
                mem_part_ref[0, h] += pms[h]

    @pl.when(j == 3)
    def _o_phase():
        out_ref[...] = jnp.dot(comb_ref[...], w_ref[0],
                               preferred_element_type=jnp.float32)


def kernel(hidden_states, Wq, Wk, Wv, Wo, bank_weights, memories, memory_norms):
    b, s, hid = hidden_states.shape
    nb, nh, d, _ = memories.shape
    M = b * s

    x = hidden_states.reshape(M, hid).astype(jnp.bfloat16)
    w_all = jnp.stack([Wq.T, Wk.T, Wv.T, Wo.T]).astype(jnp.bfloat16)

    wsoft = jax.nn.softmax(bank_weights, axis=-1)
    active = (jnp.sum(memory_norms, axis=(1, 2)) >= EPS).astype(jnp.float32)
    wb = wsoft * active[None, :]
    memsc = (memories * wb.T[:, :, None, None]).transpose(1, 0, 2, 3)
    memsc = memsc.reshape(nh, nb * d, d).astype(jnp.bfloat16)
    mem0b = memories[0].astype(jnp.bfloat16)

    eyeh = jnp.eye(nh, dtype=jnp.float32)
    nrmq = (eyeh[:, None, :, None]
            * memory_norms.transpose(1, 2, 0)[:, :, None, :])
    nrmq = nrmq.reshape(nh * d, nh * nb).astype(jnp.bfloat16)
    knrm = (eyeh[:, None, :] * memory_norms[0][:, :, None])
    knrm = knrm.reshape(nh * d, nh).astype(jnp.bfloat16)

    NC = 2
    BM = next(bm for bm in (512, 256, 128, 64, 32, 16, 8)
              if M % (NC * bm) == 0)
    NI = M // (NC * BM)

    kern = functools.partial(_phase_kernel, n_heads=nh, d_head=d, n_banks=nb)
    out_flat, mem_parts, norm_parts = pl.pallas_call(
        kern,
        grid=(NC, NI, 4),
        in_specs=[
            pl.BlockSpec((BM, hid), lambda c, i, j: (c * NI + i, 0)),
            pl.BlockSpec((1, hid, hid), lambda c, i, j: (j, 0, 0)),
            pl.BlockSpec((nh, nb * d, d), lambda c, i, j: (0, 0, 0)),
            pl.BlockSpec((nh, d, d), lambda c, i, j: (0, 0, 0)),
            pl.BlockSpec((nh * d, nh * nb), lambda c, i, j: (0, 0)),
            pl.BlockSpec((nh * d, nh), lambda c, i, j: (0, 0)),
        ],
        out_specs=[
            pl.BlockSpec((BM, hid), lambda c, i, j: (c * NI + i, 0)),
            pl.BlockSpec((1, nh, d, d), lambda c, i, j: (c, 0, 0, 0)),
            pl.BlockSpec((1, 1, hid), lambda c, i, j: (c, 0, 0)),
        ],
        out_shape=[
            jax.ShapeDtypeStruct((M, hid), jnp.float32),
            jax.ShapeDtypeStruct((NC, nh, d, d), jnp.float32),
            jax.ShapeDtypeStruct((NC, 1, hid), jnp.float32),
        ],
        scratch_shapes=[
            pltpu.VMEM((BM, hid), jnp.bfloat16),
            pltpu.VMEM((BM, hid), jnp.bfloat16),
            pltpu.VMEM((BM, hid), jnp.float32),
        ],
        compiler_params=pltpu.CompilerParams(
            dimension_semantics=("parallel", "arbitrary", "arbitrary"),
            vmem_limit_bytes=56 * 1024 * 1024,
        ),
        name="infini_attention_fused",
    )(x, w_all, memsc, mem0b, nrmq, knrm)

    out = out_flat.reshape(b, s, hid)
    new_mem0 = memories[0] + jnp.sum(mem_parts, axis=0) / (b * s)
    new_norm0 = memory_norms[0] + jnp.sum(norm_parts, axis=0).reshape(nh, d) / b
    return out, new_mem0, new_norm0


# speedup vs baseline: 2.7339x; 1.0494x over previous
import functools

import jax
import jax.numpy as jnp
from jax.experimental import pallas as pl
from jax.experimental.pallas import tpu as pltpu

EPS = 1e-6


def _phase_kernel(x_ref, w_ref, memcat_ref, mem0_ref, nrmq_ref, knrm_ref,
                  eq_ref, e16_ref,
                  out_ref, mem_part_ref, norm_part_ref,
                  comb_ref, sigk_ref, rk_ref, *, n_heads, d_head, n_banks):
    i = pl.program_id(1)
    j = pl.program_id(2)
    H, D, NB = n_heads, d_head, n_banks

    @pl.when(j == 0)
    def _q_phase():
        xb = x_ref[...].astype(jnp.bfloat16)
        q = jnp.dot(xb, w_ref[0], preferred_element_type=jnp.float32)
        sq = jnp.where(q > 0, q + 1.0, jnp.exp(q))
        sqb = sq.astype(jnp.bfloat16)
        nrm = jnp.dot(sqb, nrmq_ref[...], preferred_element_type=jnp.float32)
        invb = (1.0 / jnp.maximum(nrm, EPS)).astype(jnp.bfloat16)
        inv_exps = [
            jnp.dot(invb, eq_ref[n],
                    preferred_element_type=jnp.float32).astype(jnp.bfloat16)
            for n in range(NB)
        ]
        for h in range(H):
            u = jnp.dot(sqb[:, h * D:(h + 1) * D], memcat_ref[h],
                        preferred_element_type=jnp.float32)
            comb = u[:, 0:D] * inv_exps[0][:, h * D:(h + 1) * D]
            for n in range(1, NB):
                comb += u[:, n * D:(n + 1) * D] * inv_exps[n][:, h * D:(h + 1) * D]
            comb_ref[:, h * D:(h + 1) * D] = comb.astype(jnp.bfloat16)

    @pl.when(j == 1)
    def _k_phase():
        xb = x_ref[...].astype(jnp.bfloat16)
        k = jnp.dot(xb, w_ref[0], preferred_element_type=jnp.float32)
        sk = jnp.where(k > 0, k + 1.0, jnp.exp(k))
        skb = sk.astype(jnp.bfloat16)
        sigk_ref[...] = skb
        kn = jnp.dot(skb, knrm_ref[...], preferred_element_type=jnp.float32)
        iknb = (1.0 / jnp.maximum(kn, EPS)).astype(jnp.bfloat16)
        ikn_exp = jnp.dot(iknb, e16_ref[...],
                          preferred_element_type=jnp.float32)
        for h in range(H):
            retr = jnp.dot(skb[:, h * D:(h + 1) * D], mem0_ref[h],
                           preferred_element_type=jnp.float32)
            rk_ref[:, h * D:(h + 1) * D] = retr * ikn_exp[:, h * D:(h + 1) * D]
        part = jnp.sum(sk, axis=0)[None, None, :]

        @pl.when(i == 0)
        def _():
            norm_part_ref[...] = part

        @pl.when(i != 0)
        def _():
            norm_part_ref[...] += part

    @pl.when(j == 2)
    def _v_phase():
        xb = x_ref[...].astype(jnp.bfloat16)
        v = jnp.dot(xb, w_ref[0], preferred_element_type=jnp.float32)
        dv = (v - rk_ref[...]).astype(jnp.bfloat16)
        sk = sigk_ref[...]
        pms = [
            jax.lax.dot_general(
                sk[:, h * D:(h + 1) * D], dv[:, h * D:(h + 1) * D],
                dimension_numbers=(((0,), (0,)), ((), ())),
                preferred_element_type=jnp.float32)
            for h in range(H)
        ]

        @pl.when(i == 0)
        def _():
            for h in range(H):
                mem_part_ref[0, h] = pms[h]

        @pl.when(i != 0)
        def _():
            for h in range(H):
                mem_part_ref[0, h] += pms[h]

    @pl.when(j == 3)
    def _o_phase():
        out_ref[...] = jnp.dot(comb_ref[...], w_ref[0],
                               preferred_element_type=jnp.float32)


def kernel(hidden_states, Wq, Wk, Wv, Wo, bank_weights, memories, memory_norms):
    b, s, hid = hidden_states.shape
    nb, nh, d, _ = memories.shape
    M = b * s

    x = hidden_states.reshape(M, hid)
    w_all = jnp.stack([Wq.T, Wk.T, Wv.T, Wo.T]).astype(jnp.bfloat16)

    wsoft = jax.nn.softmax(bank_weights, axis=-1)
    active = (jnp.sum(memory_norms, axis=(1, 2)) >= EPS).astype(jnp.float32)
    wb = wsoft * active[None, :]
    memcat = (memories * wb.T[:, :, None, None]).transpose(1, 2, 0, 3)
    memcat = memcat.reshape(nh, d, nb * d).astype(jnp.bfloat16)
    mem0b = memories[0].astype(jnp.bfloat16)

    eyeh = jnp.eye(nh, dtype=jnp.float32)
    nrmq = (eyeh[:, None, :, None]
            * memory_norms.transpose(1, 2, 0)[:, :, None, :])
    nrmq = nrmq.reshape(nh * d, nh * nb).astype(jnp.bfloat16)
    knrm = (eyeh[:, None, :] * memory_norms[0][:, :, None])
    knrm = knrm.reshape(nh * d, nh).astype(jnp.bfloat16)

    eyen = jnp.eye(nb, dtype=jnp.float32)
    eq = (eyen[:, None, :, None, None] * eyeh[None, :, None, :, None])
    eq = jnp.broadcast_to(eq, (nb, nh, nb, nh, d))
    eq = eq.reshape(nb, nh * nb, nh * d).astype(jnp.bfloat16)
    e16 = jnp.broadcast_to(eyeh[:, :, None], (nh, nh, d))
    e16 = e16.reshape(nh, nh * d).astype(jnp.bfloat16)

    NC = 2
    BM = next(bm for bm in (512, 256, 128, 64, 32, 16, 8)
              if M % (NC * bm) == 0)
    NI = M // (NC * BM)

    kern = functools.partial(_phase_kernel, n_heads=nh, d_head=d, n_banks=nb)
    out_flat, mem_parts, norm_parts = pl.pallas_call(
        kern,
        grid=(NC, NI, 4),
        in_specs=[
            pl.BlockSpec((BM, hid), lambda c, i, j: (c * NI + i, 0)),
            pl.BlockSpec((1, hid, hid), lambda c, i, j: (j, 0, 0)),
            pl.BlockSpec((nh, d, nb * d), lambda c, i, j: (0, 0, 0)),
            pl.BlockSpec((nh, d, d), lambda c, i, j: (0, 0, 0)),
            pl.BlockSpec((nh * d, nh * nb), lambda c, i, j: (0, 0)),
            pl.BlockSpec((nh * d, nh), lambda c, i, j: (0, 0)),
            pl.BlockSpec((nb, nh * nb, nh * d), lambda c, i, j: (0, 0, 0)),
            pl.BlockSpec((nh, nh * d), lambda c, i, j: (0, 0)),
        ],
        out_specs=[
            pl.BlockSpec((BM, hid), lambda c, i, j: (c * NI + i, 0)),
            pl.BlockSpec((1, nh, d, d), lambda c, i, j: (c, 0, 0, 0)),
            pl.BlockSpec((1, 1, hid), lambda c, i, j: (c, 0, 0)),
        ],
        out_shape=[
            jax.ShapeDtypeStruct((M, hid), jnp.float32),
            jax.ShapeDtypeStruct((NC, nh, d, d), jnp.float32),
            jax.ShapeDtypeStruct((NC, 1, hid), jnp.float32),
        ],
        scratch_shapes=[
            pltpu.VMEM((BM, hid), jnp.bfloat16),
            pltpu.VMEM((BM, hid), jnp.bfloat16),
            pltpu.VMEM((BM, hid), jnp.float32),
        ],
        compiler_params=pltpu.CompilerParams(
            dimension_semantics=("parallel", "arbitrary", "arbitrary"),
            vmem_limit_bytes=58 * 1024 * 1024,
        ),
        name="infini_attention_fused",
    )(x, w_all, memcat, mem0b, nrmq, knrm, eq, e16)

    out = out_flat.reshape(b, s, hid)
    new_mem0 = memories[0] + jnp.sum(mem_parts, axis=0) / (b * s)
    new_norm0 = memory_norms[0] + jnp.sum(norm_parts, axis=0).reshape(nh, d) / b
    return out, new_mem0, new_norm0


# speedup vs baseline: 2.8642x; 1.0477x over previous
import functools

import jax
import jax.numpy as jnp
from jax.experimental import pallas as pl
from jax.experimental.pallas import tpu as pltpu

EPS = 1e-6


def _main_kernel(x_ref, w_hbm_ref, memcat_ref, mem0_ref, nrmq_ref, knrm_ref,
                 eq_ref, e16_ref,
                 comb_ref, mem_part_ref, norm_part_ref,
                 w_vmem, dma_sem, *, n_heads, d_head, n_banks):
    i = pl.program_id(1)
    H, D, NB = n_heads, d_head, n_banks

    @pl.when(i == 0)
    def _load_weights():
        cp = pltpu.make_async_copy(w_hbm_ref, w_vmem, dma_sem)
        cp.start()
        cp.wait()

    xb = x_ref[...].astype(jnp.bfloat16)
    q = jnp.dot(xb, w_vmem[0], preferred_element_type=jnp.float32)
    k = jnp.dot(xb, w_vmem[1], preferred_element_type=jnp.float32)
    v = jnp.dot(xb, w_vmem[2], preferred_element_type=jnp.float32)

    sq = jnp.where(q > 0, q + 1.0, jnp.exp(q))
    sqb = sq.astype(jnp.bfloat16)
    nrm = jnp.dot(sqb, nrmq_ref[...], preferred_element_type=jnp.float32)
    invb = (1.0 / jnp.maximum(nrm, EPS)).astype(jnp.bfloat16)
    inv_exps = [
        jnp.dot(invb, eq_ref[n],
                preferred_element_type=jnp.float32).astype(jnp.bfloat16)
        for n in range(NB)
    ]
    combs = []
    for h in range(H):
        u = jnp.dot(sqb[:, h * D:(h + 1) * D], memcat_ref[h],
                    preferred_element_type=jnp.float32)
        acc = u[:, 0:D] * inv_exps[0][:, h * D:(h + 1) * D]
        for n in range(1, NB):
            acc += u[:, n * D:(n + 1) * D] * inv_exps[n][:, h * D:(h + 1) * D]
        combs.append(acc.astype(jnp.bfloat16))
    comb_ref[...] = jnp.concatenate(combs, axis=1)

    sk = jnp.where(k > 0, k + 1.0, jnp.exp(k))
    skb = sk.astype(jnp.bfloat16)
    kn = jnp.dot(skb, knrm_ref[...], preferred_element_type=jnp.float32)
    iknb = (1.0 / jnp.maximum(kn, EPS)).astype(jnp.bfloat16)
    ikn_exp = jnp.dot(iknb, e16_ref[...],
                      preferred_element_type=jnp.float32)
    rks = []
    for h in range(H):
        retr = jnp.dot(skb[:, h * D:(h + 1) * D], mem0_ref[h],
                       preferred_element_type=jnp.float32)
        rks.append(retr * ikn_exp[:, h * D:(h + 1) * D])
    dv = (v - jnp.concatenate(rks, axis=1)).astype(jnp.bfloat16)
    pms = [
        jax.lax.dot_general(
            skb[:, h * D:(h + 1) * D], dv[:, h * D:(h + 1) * D],
            dimension_numbers=(((0,), (0,)), ((), ())),
            preferred_element_type=jnp.float32)
        for h in range(H)
    ]
    part = jnp.sum(sk, axis=0)[None, None, :]

    @pl.when(i == 0)
    def _init():
        for h in range(H):
            mem_part_ref[0, h] = pms[h]
        norm_part_ref[...] = part

    @pl.when(i != 0)
    def _acc():
        for h in range(H):
            mem_part_ref[0, h] += pms[h]
        norm_part_ref[...] += part


def _oproj_kernel(c_ref, w_ref, o_ref):
    o_ref[...] = jnp.dot(c_ref[...], w_ref[...],
                         preferred_element_type=jnp.float32)


def kernel(hidden_states, Wq, Wk, Wv, Wo, bank_weights, memories, memory_norms):
    b, s, hid = hidden_states.shape
    nb, nh, d, _ = memories.shape
    M = b * s

    x = hidden_states.reshape(M, hid)
    wqkv = jnp.stack([Wq.T, Wk.T, Wv.T]).astype(jnp.bfloat16)
    wo_t = Wo.T.astype(jnp.bfloat16)

    wsoft = jax.nn.softmax(bank_weights, axis=-1)
    active = (jnp.sum(memory_norms, axis=(1, 2)) >= EPS).astype(jnp.float32)
    wb = wsoft * active[None, :]
    memcat = (memories * wb.T[:, :, None, None]).transpose(1, 2, 0, 3)
    memcat = memcat.reshape(nh, d, nb * d).astype(jnp.bfloat16)
    mem0b = memories[0].astype(jnp.bfloat16)

    eyeh = jnp.eye(nh, dtype=jnp.float32)
    nrmq = (eyeh[:, None, :, None]
            * memory_norms.transpose(1, 2, 0)[:, :, None, :])
    nrmq = nrmq.reshape(nh * d, nh * nb).astype(jnp.bfloat16)
    knrm = (eyeh[:, None, :] * memory_norms[0][:, :, None])
    knrm = knrm.reshape(nh * d, nh).astype(jnp.bfloat16)

    eyen = jnp.eye(nb, dtype=jnp.float32)
    eq = (eyen[:, None, :, None, None] * eyeh[None, :, None, :, None])
    eq = jnp.broadcast_to(eq, (nb, nh, nb, nh, d))
    eq = eq.reshape(nb, nh * nb, nh * d).astype(jnp.bfloat16)
    e16 = jnp.broadcast_to(eyeh[:, :, None], (nh, nh, d))
    e16 = e16.reshape(nh, nh * d).astype(jnp.bfloat16)

    NC = 2
    BM = next(bm for bm in (256, 128, 64, 32, 16, 8)
              if M % (NC * bm) == 0)
    NI = M // (NC * BM)

    kern = functools.partial(_main_kernel, n_heads=nh, d_head=d, n_banks=nb)
    comb, mem_parts, norm_parts = pl.pallas_call(
        kern,
        grid=(NC, NI),
        in_specs=[
            pl.BlockSpec((BM, hid), lambda c, i: (c * NI + i, 0)),
            pl.BlockSpec(memory_space=pl.ANY),
            pl.BlockSpec((nh, d, nb * d), lambda c, i: (0, 0, 0)),
            pl.BlockSpec((nh, d, d), lambda c, i: (0, 0, 0)),
            pl.BlockSpec((nh * d, nh * nb), lambda c, i: (0, 0)),
            pl.BlockSpec((nh * d, nh), lambda c, i: (0, 0)),
            pl.BlockSpec((nb, nh * nb, nh * d), lambda c, i: (0, 0, 0)),
            pl.BlockSpec((nh, nh * d), lambda c, i: (0, 0)),
        ],
        out_specs=[
            pl.BlockSpec((BM, hid), lambda c, i: (c * NI + i, 0)),
            pl.BlockSpec((1, nh, d, d), lambda c, i: (c, 0, 0, 0)),
            pl.BlockSpec((1, 1, hid), lambda c, i: (c, 0, 0)),
        ],
        out_shape=[
            jax.ShapeDtypeStruct((M, hid), jnp.bfloat16),
            jax.ShapeDtypeStruct((NC, nh, d, d), jnp.float32),
            jax.ShapeDtypeStruct((NC, 1, hid), jnp.float32),
        ],
        scratch_shapes=[
            pltpu.VMEM((3, hid, hid), jnp.bfloat16),
            pltpu.SemaphoreType.DMA,
        ],
        compiler_params=pltpu.CompilerParams(
            dimension_semantics=("parallel", "arbitrary"),
            vmem_limit_bytes=58 * 1024 * 1024,
        ),
        name="infini_attention_main",
    )(x, wqkv, memcat, mem0b, nrmq, knrm, eq, e16)

    BO = 1024 if M % 1024 == 0 else BM
    out_flat = pl.pallas_call(
        _oproj_kernel,
        grid=(M // BO,),
        in_specs=[
            pl.BlockSpec((BO, hid), lambda i: (i, 0)),
            pl.BlockSpec((hid, hid), lambda i: (0, 0)),
        ],
        out_specs=pl.BlockSpec((BO, hid), lambda i: (i, 0)),
        out_shape=jax.ShapeDtypeStruct((M, hid), jnp.float32),
        compiler_params=pltpu.CompilerParams(
            dimension_semantics=("arbitrary",),
            vmem_limit_bytes=48 * 1024 * 1024,
        ),
        name="infini_attention_oproj",
    )(comb, wo_t)

    out = out_flat.reshape(b, s, hid)
    new_mem0 = memories[0] + jnp.sum(mem_parts, axis=0) / (b * s)
    new_norm0 = memory_norms[0] + jnp.sum(norm_parts, axis=0).reshape(nh, d) / b
    return out, new_mem0, new_norm0
